# baseline (speedup 1.0000x reference)
import functools

import jax
import jax.numpy as jnp
from jax import lax
from jax.experimental import pallas as pl
import jax.experimental.pallas.tpu as pltpu
from jax.experimental.pallas import tpu_sc as plsc

B, S, D, E = 4, 2048, 4096, 64

R_SC = 384
S_TC = S - R_SC
S_CHUNK = 128
N_SCHUNKS = S_TC // S_CHUNK

K_CHUNK = 512
N_KCHUNKS = D // K_CHUNK
PH2 = 1 + N_KCHUNKS
NSTEPS2 = 1 + 2 * N_KCHUNKS

NW = 32
DW = D // NW
L = 16
RG = 16
N_GROUPS = R_SC // RG


def _sc_partial_sum(state_hbm, out_hbm, buf0, buf1, outbuf, sem0, sem1):
    wid = lax.axis_index("s") * 2 + lax.axis_index("c")
    col = wid * DW
    bufs = (buf0, buf1)
    sems = (sem0, sem1)
    handles = [None] * B
    handles[0] = pltpu.async_copy(
        state_hbm.at[0, pl.ds(S_TC, R_SC), pl.ds(col, DW)], buf0, sem0)
    for b in range(B):
        if b + 1 < B:
            handles[b + 1] = pltpu.async_copy(
                state_hbm.at[b + 1, pl.ds(S_TC, R_SC), pl.ds(col, DW)],
                bufs[(b + 1) % 2], sems[(b + 1) % 2])
        handles[b].wait()
        cur = bufs[b % 2]

        def grp(gi, accs, cur=cur):
            base = gi * RG
            out = []
            for j in range(DW // L):
                v = accs[j]
                for r in range(RG):
                    v = v + cur[base + r, pl.ds(L * j, L)]
                out.append(v)
            return out

        accs = lax.fori_loop(
            0, N_GROUPS, grp,
            [jnp.zeros((L,), jnp.float32) for _ in range(DW // L)])
        for j in range(DW // L):
            outbuf[pl.ds(L * j, L)] = accs[j]
        pltpu.sync_copy(outbuf, out_hbm.at[b, pl.ds(col, DW)])


def _tc1_kernel(state_ref, part_ref, acc_ref):
    i = pl.program_id(0)

    @pl.when(i == 0)
    def _init():
        acc_ref[...] = jnp.zeros_like(acc_ref)

    acc_ref[...] += jnp.sum(state_ref[...], axis=1)

    @pl.when(i == N_SCHUNKS - 1)
    def _finish():
        part_ref[...] = acc_ref[...]


def _tc2_kernel(ptc_ref, psc_ref, scale_ref, bias_ref, w1_ref, b1_ref,
                w2_ref, b2_ref, emb_ref, cb_ref, probs_ref,
                h_ref, zacc_ref, g_ref):
    i = pl.program_id(0)

    @pl.when(i == 0)
    def _layernorm():
        m = (ptc_ref[...] + psc_ref[...]) * (1.0 / S)
        mu = jnp.mean(m, axis=-1, keepdims=True)
        var = jnp.mean((m - mu) ** 2, axis=-1, keepdims=True)
        h = (m - mu) * jax.lax.rsqrt(var + 1e-5)
        h_ref[...] = h * scale_ref[...] + bias_ref[...]
        zacc_ref[...] = jnp.zeros_like(zacc_ref)

    @pl.when((i >= 1) & (i < PH2))
    def _mlp1_phase():
        hk = h_ref[:, pl.ds((i - 1) * K_CHUNK, K_CHUNK)]
        zacc_ref[...] += jnp.dot(hk, w1_ref[...],
                                 preferred_element_type=jnp.float32)

    @pl.when(i == PH2 - 1)
    def _gelu():
        z = zacc_ref[...] + b1_ref[...]
        g_ref[...] = z * 0.5 * (1.0 + jax.lax.erf(z * 0.7071067811865476))
        zacc_ref[...] = jnp.zeros_like(zacc_ref)

    @pl.when(i >= PH2)
    def _mlp2_phase():
        gk = g_ref[:, pl.ds((i - PH2) * K_CHUNK, K_CHUNK)]
        zacc_ref[...] += jnp.dot(gk, w2_ref[...],
                                 preferred_element_type=jnp.float32)

    @pl.when(i == NSTEPS2 - 1)
    def _finish():
        p = zacc_ref[...] + b2_ref[...]
        att = jnp.dot(p, emb_ref[...].T, preferred_element_type=jnp.float32)
        att = att + cb_ref[...]
        att = att - jnp.max(att, axis=-1, keepdims=True)
        ex = jnp.exp(att)
        probs_ref[...] = ex / jnp.sum(ex, axis=-1, keepdims=True)


def _clamp(lo, x, hi):
    return jnp.minimum(jnp.maximum(x, lo), hi)


def _make_sc_sum():
    return pl.kernel(
        _sc_partial_sum,
        mesh=plsc.VectorSubcoreMesh(core_axis_name="c", subcore_axis_name="s"),
        out_type=jax.ShapeDtypeStruct((B, D), jnp.float32),
        scratch_types=[
            pltpu.VMEM((R_SC, DW), jnp.float32),
            pltpu.VMEM((R_SC, DW), jnp.float32),
            pltpu.VMEM((DW,), jnp.float32),
            pltpu.SemaphoreType.DMA,
            pltpu.SemaphoreType.DMA,
        ],
    )


def kernel(state, ln_scale, ln_bias, W1, b1, W2, b2, expert_emb, centrality, spatial, current_expert_idx):
    scale2 = ln_scale.reshape(1, D)
    bias2 = ln_bias.reshape(1, D)
    b1_2 = b1.reshape(1, D)
    b2_2 = b2.reshape(1, D)
    spatial_row = jax.lax.dynamic_index_in_dim(spatial, current_expert_idx, 0, keepdims=False)
    combined_bias = (centrality + spatial_row).reshape(1, E)

    part_sc = _make_sc_sum()(state)

    part_tc = pl.pallas_call(
        _tc1_kernel,
        grid=(N_SCHUNKS,),
        in_specs=[
            pl.BlockSpec((B, S_CHUNK, D), lambda i: (0, i, 0)),
        ],
        out_specs=pl.BlockSpec((B, D), lambda i: (0, 0)),
        out_shape=jax.ShapeDtypeStruct((B, D), jnp.float32),
        scratch_shapes=[pltpu.VMEM((B, D), jnp.float32)],
    )(state)

    probs = pl.pallas_call(
        _tc2_kernel,
        grid=(NSTEPS2,),
        in_specs=[
            pl.BlockSpec((B, D), lambda i: (0, 0)),
            pl.BlockSpec((B, D), lambda i: (0, 0)),
            pl.BlockSpec((1, D), lambda i: (0, 0)),
            pl.BlockSpec((1, D), lambda i: (0, 0)),
            pl.BlockSpec((K_CHUNK, D), lambda i: (_clamp(0, i - 1, N_KCHUNKS - 1), 0)),
            pl.BlockSpec((1, D), lambda i: (0, 0)),
            pl.BlockSpec((K_CHUNK, D), lambda i: (_clamp(0, i - PH2, N_KCHUNKS - 1), 0)),
            pl.BlockSpec((1, D), lambda i: (0, 0)),
            pl.BlockSpec((E, D), lambda i: (0, 0)),
            pl.BlockSpec((1, E), lambda i: (0, 0)),
        ],
        out_specs=pl.BlockSpec((B, E), lambda i: (0, 0)),
        out_shape=jax.ShapeDtypeStruct((B, E), jnp.float32),
        scratch_shapes=[
            pltpu.VMEM((B, D), jnp.float32),
            pltpu.VMEM((B, D), jnp.float32),
            pltpu.VMEM((B, D), jnp.float32),
        ],
    )(part_tc, part_sc, scale2, bias2, W1, b1_2, W2, b2_2, expert_emb, combined_bias)

    return probs


# speedup vs baseline: 1.0045x; 1.0045x over previous
import functools

import jax
import jax.numpy as jnp
from jax import lax
from jax.experimental import pallas as pl
import jax.experimental.pallas.tpu as pltpu
from jax.experimental.pallas import tpu_sc as plsc

B, S, D, E = 4, 2048, 4096, 64

R_SC = 384
S_TC = S - R_SC
S_CHUNK = 128
N_SCHUNKS = S_TC // S_CHUNK

K_CHUNK = 512
N_KCHUNKS = D // K_CHUNK
PH2 = 1 + N_KCHUNKS
NSTEPS2 = 1 + 2 * N_KCHUNKS

NW = 32
DW = D // NW
L = 16
RG = 16
N_GROUPS = R_SC // RG


def _sc_partial_sum(state_hbm, out_hbm, buf0, buf1, outbuf, sem0, sem1):
    wid = lax.axis_index("s") * 2 + lax.axis_index("c")
    col = wid * DW
    bufs = (buf0, buf1)
    sems = (sem0, sem1)
    handles = [None] * B
    handles[0] = pltpu.async_copy(
        state_hbm.at[0, pl.ds(S_TC, R_SC), pl.ds(col, DW)], buf0, sem0)
    for b in range(B):
        if b + 1 < B:
            handles[b + 1] = pltpu.async_copy(
                state_hbm.at[b + 1, pl.ds(S_TC, R_SC), pl.ds(col, DW)],
                bufs[(b + 1) % 2], sems[(b + 1) % 2])
        handles[b].wait()
        cur = bufs[b % 2]

        def grp(gi, accs, cur=cur):
            base = gi * RG
            out = []
            for j in range(DW // L):
                v = accs[j]
                for r in range(RG):
                    v = v + cur[base + r, pl.ds(L * j, L)]
                out.append(v)
            return out

        accs = lax.fori_loop(
            0, N_GROUPS, grp,
            [jnp.zeros((L,), jnp.float32) for _ in range(DW // L)])
        for j in range(DW // L):
            outbuf[pl.ds(L * j, L)] = accs[j]
        pltpu.sync_copy(outbuf, out_hbm.at[b, pl.ds(col, DW)])


def _tc1_kernel(state_ref, part_ref, acc_ref):
    i = pl.program_id(0)

    @pl.when(i == 0)
    def _init():
        acc_ref[...] = jnp.zeros_like(acc_ref)

    acc_ref[...] += jnp.sum(state_ref[...], axis=1)

    @pl.when(i == N_SCHUNKS - 1)
    def _finish():
        part_ref[...] = acc_ref[...]


def _tc2_kernel(ptc_ref, psc_ref, scale_ref, bias_ref, w1_ref, b1_ref,
                w2_ref, b2_ref, emb_ref, cb_ref, probs_ref,
                h_ref, zacc_ref, g_ref):
    i = pl.program_id(0)

    @pl.when(i == 0)
    def _layernorm():
        m = (ptc_ref[...] + psc_ref[...]) * (1.0 / S)
        mu = jnp.mean(m, axis=-1, keepdims=True)
        var = jnp.mean((m - mu) ** 2, axis=-1, keepdims=True)
        h = (m - mu) * jax.lax.rsqrt(var + 1e-5)
        h_ref[...] = h * scale_ref[...] + bias_ref[...]
        zacc_ref[...] = jnp.zeros_like(zacc_ref)

    @pl.when((i >= 1) & (i < PH2))
    def _mlp1_phase():
        hk = h_ref[:, pl.ds((i - 1) * K_CHUNK, K_CHUNK)]
        zacc_ref[...] += jnp.dot(hk, w1_ref[...],
                                 preferred_element_type=jnp.float32)

    @pl.when(i == PH2 - 1)
    def _gelu():
        z = zacc_ref[...] + b1_ref[...]
        g_ref[...] = z * 0.5 * (1.0 + jax.lax.erf(z * 0.7071067811865476))
        zacc_ref[...] = jnp.zeros_like(zacc_ref)

    @pl.when(i >= PH2)
    def _mlp2_phase():
        gk = g_ref[:, pl.ds((i - PH2) * K_CHUNK, K_CHUNK)]
        zacc_ref[...] += jnp.dot(gk, w2_ref[...],
                                 preferred_element_type=jnp.float32)

    @pl.when(i == NSTEPS2 - 1)
    def _finish():
        p = zacc_ref[...] + b2_ref[...]
        att = jnp.dot(p, emb_ref[...].T, preferred_element_type=jnp.float32)
        att = att + cb_ref[...]
        att = att - jnp.max(att, axis=-1, keepdims=True)
        ex = jnp.exp(att)
        probs_ref[...] = ex / jnp.sum(ex, axis=-1, keepdims=True)


def _clamp(lo, x, hi):
    return jnp.minimum(jnp.maximum(x, lo), hi)


def _make_sc_sum():
    return pl.kernel(
        _sc_partial_sum,
        mesh=plsc.VectorSubcoreMesh(core_axis_name="c", subcore_axis_name="s"),
        out_type=jax.ShapeDtypeStruct((B, D), jnp.float32),
        scratch_types=[
            pltpu.VMEM((R_SC, DW), jnp.float32),
            pltpu.VMEM((R_SC, DW), jnp.float32),
            pltpu.VMEM((DW,), jnp.float32),
            pltpu.SemaphoreType.DMA,
            pltpu.SemaphoreType.DMA,
        ],
    )


def kernel(state, ln_scale, ln_bias, W1, b1, W2, b2, expert_emb, centrality, spatial, current_expert_idx):
    scale2 = ln_scale.reshape(1, D)
    bias2 = ln_bias.reshape(1, D)
    b1_2 = b1.reshape(1, D)
    b2_2 = b2.reshape(1, D)
    spatial_row = jax.lax.dynamic_index_in_dim(spatial, current_expert_idx, 0, keepdims=False)
    combined_bias = (centrality + spatial_row).reshape(1, E)

    part_tc = pl.pallas_call(
        _tc1_kernel,
        grid=(N_SCHUNKS,),
        in_specs=[
            pl.BlockSpec((B, S_CHUNK, D), lambda i: (0, i, 0)),
        ],
        out_specs=pl.BlockSpec((B, D), lambda i: (0, 0)),
        out_shape=jax.ShapeDtypeStruct((B, D), jnp.float32),
        scratch_shapes=[pltpu.VMEM((B, D), jnp.float32)],
    )(state)

    part_sc = _make_sc_sum()(state)

    probs = pl.pallas_call(
        _tc2_kernel,
        grid=(NSTEPS2,),
        in_specs=[
            pl.BlockSpec((B, D), lambda i: (0, 0)),
            pl.BlockSpec((B, D), lambda i: (0, 0)),
            pl.BlockSpec((1, D), lambda i: (0, 0)),
            pl.BlockSpec((1, D), lambda i: (0, 0)),
            pl.BlockSpec((K_CHUNK, D), lambda i: (_clamp(0, i - 1, N_KCHUNKS - 1), 0)),
            pl.BlockSpec((1, D), lambda i: (0, 0)),
            pl.BlockSpec((K_CHUNK, D), lambda i: (_clamp(0, i - PH2, N_KCHUNKS - 1), 0)),
            pl.BlockSpec((1, D), lambda i: (0, 0)),
            pl.BlockSpec((E, D), lambda i: (0, 0)),
            pl.BlockSpec((1, E), lambda i: (0, 0)),
        ],
        out_specs=pl.BlockSpec((B, E), lambda i: (0, 0)),
        out_shape=jax.ShapeDtypeStruct((B, E), jnp.float32),
        scratch_shapes=[
            pltpu.VMEM((B, D), jnp.float32),
            pltpu.VMEM((B, D), jnp.float32),
            pltpu.VMEM((B, D), jnp.float32),
        ],
    )(part_tc, part_sc, scale2, bias2, W1, b1_2, W2, b2_2, expert_emb, combined_bias)

    return probs


# speedup vs baseline: 1.2075x; 1.2021x over previous
import jax
import jax.numpy as jnp
from jax.experimental import pallas as pl
import jax.experimental.pallas.tpu as pltpu

B, S, D, E = 4, 2048, 4096, 64
S_CHUNK = 128
N_SCHUNKS = S // S_CHUNK
K_CHUNK = 512
N_KCHUNKS = D // K_CHUNK
PH1 = N_SCHUNKS
PH2 = N_SCHUNKS + N_KCHUNKS
NSTEPS = N_SCHUNKS + 2 * N_KCHUNKS


def _fused_kernel(state_ref, scale_ref, bias_ref, w1_ref, b1_ref,
                  w2_ref, b2_ref, emb_ref, cb_ref, probs_ref,
                  macc_ref, h_ref, zacc_ref, g_ref):
    i = pl.program_id(0)

    @pl.when(i == 0)
    def _init():
        macc_ref[...] = jnp.zeros_like(macc_ref)

    @pl.when(i < PH1)
    def _mean_phase():
        macc_ref[...] += jnp.sum(state_ref[...], axis=1)

    @pl.when(i == PH1 - 1)
    def _layernorm():
        m = macc_ref[...] * (1.0 / S)
        mu = jnp.mean(m, axis=-1, keepdims=True)
        var = jnp.mean((m - mu) ** 2, axis=-1, keepdims=True)
        h = (m - mu) * jax.lax.rsqrt(var + 1e-5)
        h_ref[...] = h * scale_ref[...] + bias_ref[...]
        zacc_ref[...] = jnp.zeros_like(zacc_ref)

    @pl.when((i >= PH1) & (i < PH2))
    def _mlp1_phase():
        hk = h_ref[:, pl.ds((i - PH1) * K_CHUNK, K_CHUNK)]
        zacc_ref[...] += jnp.dot(hk, w1_ref[...],
                                 preferred_element_type=jnp.float32)

    @pl.when(i == PH2 - 1)
    def _gelu():
        z = zacc_ref[...] + b1_ref[...]
        g_ref[...] = z * 0.5 * (1.0 + jax.lax.erf(z * 0.7071067811865476))
        zacc_ref[...] = jnp.zeros_like(zacc_ref)

    @pl.when(i >= PH2)
    def _mlp2_phase():
        gk = g_ref[:, pl.ds((i - PH2) * K_CHUNK, K_CHUNK)]
        zacc_ref[...] += jnp.dot(gk, w2_ref[...],
                                 preferred_element_type=jnp.float32)

    @pl.when(i == NSTEPS - 1)
    def _finish():
        p = zacc_ref[...] + b2_ref[...]
        att = jnp.dot(p, emb_ref[...].T, preferred_element_type=jnp.float32)
        att = att + cb_ref[...]
        att = att - jnp.max(att, axis=-1, keepdims=True)
        ex = jnp.exp(att)
        probs_ref[...] = ex / jnp.sum(ex, axis=-1, keepdims=True)


def _clamp(lo, x, hi):
    return jnp.minimum(jnp.maximum(x, lo), hi)


def kernel(state, ln_scale, ln_bias, W1, b1, W2, b2, expert_emb, centrality, spatial, current_expert_idx):
    scale2 = ln_scale.reshape(1, D)
    bias2 = ln_bias.reshape(1, D)
    b1_2 = b1.reshape(1, D)
    b2_2 = b2.reshape(1, D)
    spatial_row = jax.lax.dynamic_index_in_dim(spatial, current_expert_idx, 0, keepdims=False)
    combined_bias = (centrality + spatial_row).reshape(1, E)

    probs = pl.pallas_call(
        _fused_kernel,
        grid=(NSTEPS,),
        in_specs=[
            pl.BlockSpec((B, S_CHUNK, D), lambda i: (0, jnp.minimum(i, N_SCHUNKS - 1), 0)),
            pl.BlockSpec((1, D), lambda i: (0, 0)),
            pl.BlockSpec((1, D), lambda i: (0, 0)),
            pl.BlockSpec((K_CHUNK, D), lambda i: (_clamp(0, i - PH1, N_KCHUNKS - 1), 0)),
            pl.BlockSpec((1, D), lambda i: (0, 0)),
            pl.BlockSpec((K_CHUNK, D), lambda i: (_clamp(0, i - PH2, N_KCHUNKS - 1), 0)),
            pl.BlockSpec((1, D), lambda i: (0, 0)),
            pl.BlockSpec((E, D), lambda i: (0, 0)),
            pl.BlockSpec((1, E), lambda i: (0, 0)),
        ],
        out_specs=pl.BlockSpec((B, E), lambda i: (0, 0)),
        out_shape=jax.ShapeDtypeStruct((B, E), jnp.float32),
        scratch_shapes=[
            pltpu.VMEM((B, D), jnp.float32),
            pltpu.VMEM((B, D), jnp.float32),
            pltpu.VMEM((B, D), jnp.float32),
            pltpu.VMEM((B, D), jnp.float32),
        ],
    )(state, scale2, bias2, W1, b1_2, W2, b2_2, expert_emb, combined_bias)

    return probs


# speedup vs baseline: 1.2538x; 1.0383x over previous
import jax
import jax.numpy as jnp
from jax.experimental import pallas as pl
import jax.experimental.pallas.tpu as pltpu

B, S, D, E = 4, 2048, 4096, 64
S_CHUNK = 128
N_SCHUNKS = S // S_CHUNK
K_CHUNK = 512
N_KCHUNKS = D // K_CHUNK
LN_STEP = N_SCHUNKS - 1
GELU_STEP = LN_STEP + N_KCHUNKS - 1
NSTEPS = GELU_STEP + N_KCHUNKS


def _w1_chunk(i):
    return jnp.minimum(N_KCHUNKS - 1, jnp.maximum(0, GELU_STEP - i))


def _w2_chunk(i):
    return jnp.minimum(N_KCHUNKS - 1, jnp.maximum(0, NSTEPS - 1 - i))


def _fused_kernel(state_ref, scale_ref, bias_ref, w1_ref, b1_ref,
                  w2_ref, b2_ref, emb_ref, cb_ref, probs_ref,
                  macc_ref, h_ref, zacc_ref, g_ref):
    i = pl.program_id(0)

    @pl.when(i == 0)
    def _init():
        macc_ref[...] = jnp.zeros_like(macc_ref)

    @pl.when(i < N_SCHUNKS)
    def _mean_phase():
        macc_ref[...] += jnp.sum(state_ref[...], axis=1)

    @pl.when(i == LN_STEP)
    def _layernorm():
        m = macc_ref[...] * (1.0 / S)
        mu = jnp.mean(m, axis=-1, keepdims=True)
        var = jnp.mean((m - mu) ** 2, axis=-1, keepdims=True)
        h = (m - mu) * jax.lax.rsqrt(var + 1e-5)
        h_ref[...] = h * scale_ref[...] + bias_ref[...]
        zacc_ref[...] = jnp.zeros_like(zacc_ref)

    @pl.when((i >= LN_STEP) & (i <= GELU_STEP))
    def _mlp1_phase():
        kc = _w1_chunk(i)
        hk = h_ref[:, pl.ds(kc * K_CHUNK, K_CHUNK)]
        zacc_ref[...] += jnp.dot(hk, w1_ref[...],
                                 preferred_element_type=jnp.float32)

    @pl.when(i == GELU_STEP)
    def _gelu():
        z = zacc_ref[...] + b1_ref[...]
        g_ref[...] = z * 0.5 * (1.0 + jax.lax.erf(z * 0.7071067811865476))
        zacc_ref[...] = jnp.zeros_like(zacc_ref)

    @pl.when(i >= GELU_STEP)
    def _mlp2_phase():
        kc = _w2_chunk(i)
        gk = g_ref[:, pl.ds(kc * K_CHUNK, K_CHUNK)]
        zacc_ref[...] += jnp.dot(gk, w2_ref[...],
                                 preferred_element_type=jnp.float32)

    @pl.when(i == NSTEPS - 1)
    def _finish():
        p = zacc_ref[...] + b2_ref[...]
        att = jnp.dot(p, emb_ref[...].T, preferred_element_type=jnp.float32)
        att = att + cb_ref[...]
        att = att - jnp.max(att, axis=-1, keepdims=True)
        ex = jnp.exp(att)
        probs_ref[...] = ex / jnp.sum(ex, axis=-1, keepdims=True)


def kernel(state, ln_scale, ln_bias, W1, b1, W2, b2, expert_emb, centrality, spatial, current_expert_idx):
    scale2 = ln_scale.reshape(1, D)
    bias2 = ln_bias.reshape(1, D)
    b1_2 = b1.reshape(1, D)
    b2_2 = b2.reshape(1, D)
    spatial_row = jax.lax.dynamic_index_in_dim(spatial, current_expert_idx, 0, keepdims=False)
    combined_bias = (centrality + spatial_row).reshape(1, E)

    probs = pl.pallas_call(
        _fused_kernel,
        grid=(NSTEPS,),
        in_specs=[
            pl.BlockSpec((B, S_CHUNK, D), lambda i: (0, jnp.minimum(i, N_SCHUNKS - 1), 0)),
            pl.BlockSpec((1, D), lambda i: (0, 0)),
            pl.BlockSpec((1, D), lambda i: (0, 0)),
            pl.BlockSpec((K_CHUNK, D), lambda i: (_w1_chunk(i), 0)),
            pl.BlockSpec((1, D), lambda i: (0, 0)),
            pl.BlockSpec((K_CHUNK, D), lambda i: (_w2_chunk(i), 0)),
            pl.BlockSpec((1, D), lambda i: (0, 0)),
            pl.BlockSpec((E, D), lambda i: (0, 0)),
            pl.BlockSpec((1, E), lambda i: (0, 0)),
        ],
        out_specs=pl.BlockSpec((B, E), lambda i: (0, 0)),
        out_shape=jax.ShapeDtypeStruct((B, E), jnp.float32),
        scratch_shapes=[
            pltpu.VMEM((B, D), jnp.float32),
            pltpu.VMEM((B, D), jnp.float32),
            pltpu.VMEM((B, D), jnp.float32),
            pltpu.VMEM((B, D), jnp.float32),
        ],
    )(state, scale2, bias2, W1, b1_2, W2, b2_2, expert_emb, combined_bias)

    return probs
